# Initial kernel scaffold; baseline (speedup 1.0000x reference)
import jax
import jax.numpy as jnp
from jax.experimental import pallas as pl


def kernel(x, pe, time_indices):
    raise NotImplementedError("write your pallas kernel here")


# speedup vs baseline: 1.3890x; 1.3890x over previous
import functools

import jax
import jax.numpy as jnp
from jax import lax
from jax.experimental import pallas as pl
from jax.experimental.pallas import tpu as pltpu
from jax.experimental.pallas import tpu_sc as plsc


def _sc_gather(pe, idx, chunk=32):
    N = idx.shape[0]
    V, D = pe.shape
    info = plsc.get_sparse_core_info()
    NC, NS = info.num_cores, info.num_subcores
    NW = NC * NS
    per_w = N // NW
    n_chunks = per_w // chunk
    assert n_chunks % 2 == 0
    mesh = plsc.VectorSubcoreMesh(core_axis_name="c", subcore_axis_name="s")

    @functools.partial(
        pl.kernel,
        mesh=mesh,
        out_type=jax.ShapeDtypeStruct((N, D), jnp.float32),
        scratch_types=[
            pltpu.VMEM((per_w,), jnp.int32),
            pltpu.VMEM((2, chunk, D), jnp.float32),
            pltpu.SemaphoreType.DMA,
            pltpu.SemaphoreType.DMA,
        ],
    )
    def k(pe_hbm, idx_hbm, out_hbm, idx_v, rows_v, gsem, wsem):
        sems = (gsem, wsem)
        wid = lax.axis_index("s") * NC + lax.axis_index("c")
        base = wid * per_w
        pltpu.sync_copy(idx_hbm.at[pl.ds(base, per_w)], idx_v)

        def gather_start(c, buf):
            pltpu.async_copy(
                pe_hbm.at[idx_v.at[pl.ds(c * chunk, chunk)]],
                rows_v.at[buf],
                sems[buf],
            )

        def gather_wait(buf):
            pltpu.make_async_copy(
                pe_hbm.at[idx_v.at[pl.ds(0, chunk)]], rows_v.at[buf], sems[buf]
            ).wait()

        def write(c, buf):
            pltpu.sync_copy(rows_v.at[buf], out_hbm.at[pl.ds(base + c * chunk, chunk)])

        gather_start(0, 0)
        gather_start(1, 1)

        def body(g, _):
            c0 = 2 * g
            gather_wait(0)
            write(c0, 0)

            @pl.when(c0 + 2 < n_chunks)
            def _():
                gather_start(c0 + 2, 0)

            gather_wait(1)
            write(c0 + 1, 1)

            @pl.when(c0 + 3 < n_chunks)
            def _():
                gather_start(c0 + 3, 1)

            return 0

        lax.fori_loop(0, n_chunks // 2, body, 0)

    return k(pe, idx)


def _tc_transpose_add(x, enc_bsd, s_blk=512):
    B, D, S = x.shape

    def body(enc_ref, x_ref, o_ref):
        o_ref[0] = lax.transpose(enc_ref[0], (1, 0)) + x_ref[0]

    return pl.pallas_call(
        body,
        grid=(B, S // s_blk),
        in_specs=[
            pl.BlockSpec((1, s_blk, D), lambda b, s: (b, s, 0)),
            pl.BlockSpec((1, D, s_blk), lambda b, s: (b, 0, s)),
        ],
        out_specs=pl.BlockSpec((1, D, s_blk), lambda b, s: (b, 0, s)),
        out_shape=jax.ShapeDtypeStruct((B, D, S), jnp.float32),
    )(enc_bsd, x)


def kernel(x, pe, time_indices):
    B, D, S = x.shape
    idx = time_indices.reshape(B * S)
    enc = _sc_gather(pe, idx)
    enc = enc.reshape(B, S, D)
    return _tc_transpose_add(x, enc)


# speedup vs baseline: 1.4249x; 1.0259x over previous
import functools

import jax
import jax.numpy as jnp
from jax import lax
from jax.experimental import pallas as pl
from jax.experimental.pallas import tpu as pltpu
from jax.experimental.pallas import tpu_sc as plsc


def _sc_gather(pe, idx, chunk=32):
    N = idx.shape[0]
    V, D = pe.shape
    info = plsc.get_sparse_core_info()
    NC, NS = info.num_cores, info.num_subcores
    NW = NC * NS
    per_w = N // NW
    n_chunks = per_w // chunk
    assert n_chunks % 2 == 0
    mesh = plsc.VectorSubcoreMesh(core_axis_name="c", subcore_axis_name="s")

    @functools.partial(
        pl.kernel,
        mesh=mesh,
        out_type=jax.ShapeDtypeStruct((N, D), jnp.float32),
        scratch_types=[
            pltpu.VMEM((per_w,), jnp.int32),
            pltpu.VMEM((2, chunk, D), jnp.float32),
            pltpu.SemaphoreType.DMA,
            pltpu.SemaphoreType.DMA,
        ],
    )
    def k(pe_hbm, idx_hbm, out_hbm, idx_v, rows_v, gsem, wsem):
        sems = (gsem, wsem)
        wid = lax.axis_index("s") * NC + lax.axis_index("c")
        base = wid * per_w
        pltpu.sync_copy(idx_hbm.at[pl.ds(base, per_w)], idx_v)

        def gather_start(c, buf):
            pltpu.async_copy(
                pe_hbm.at[idx_v.at[pl.ds(c * chunk, chunk)]],
                rows_v.at[buf],
                sems[buf],
            )

        def gather_wait(buf):
            pltpu.make_async_copy(
                pe_hbm.at[idx_v.at[pl.ds(0, chunk)]], rows_v.at[buf], sems[buf]
            ).wait()

        def write(c, buf):
            pltpu.sync_copy(rows_v.at[buf], out_hbm.at[pl.ds(base + c * chunk, chunk)])

        gather_start(0, 0)
        gather_start(1, 1)

        def body(g, _):
            c0 = 2 * g
            gather_wait(0)
            write(c0, 0)

            @pl.when(c0 + 2 < n_chunks)
            def _():
                gather_start(c0 + 2, 0)

            gather_wait(1)
            write(c0 + 1, 1)

            @pl.when(c0 + 3 < n_chunks)
            def _():
                gather_start(c0 + 3, 1)

            return 0

        lax.fori_loop(0, n_chunks // 2, body, 0)

    return k(pe, idx)


def _tc_transpose_add(x, enc_bsd, s_blk=1024):
    B, D, S = x.shape

    def body(enc_ref, x_ref, o_ref):
        o_ref[0] = lax.transpose(enc_ref[0], (1, 0)) + x_ref[0]

    return pl.pallas_call(
        body,
        grid=(B, S // s_blk),
        in_specs=[
            pl.BlockSpec((1, s_blk, D), lambda b, s: (b, s, 0)),
            pl.BlockSpec((1, D, s_blk), lambda b, s: (b, 0, s)),
        ],
        out_specs=pl.BlockSpec((1, D, s_blk), lambda b, s: (b, 0, s)),
        out_shape=jax.ShapeDtypeStruct((B, D, S), jnp.float32),
    )(enc_bsd, x)


def kernel(x, pe, time_indices):
    B, D, S = x.shape
    idx = time_indices.reshape(B * S)
    enc = _sc_gather(pe, idx)
    enc = enc.reshape(B, S, D)
    return _tc_transpose_add(x, enc)
